# baseline (speedup 1.0000x reference)
import jax
import jax.numpy as jnp
from jax.experimental import pallas as pl


B, N, KD, VD = 4096, 200, 64, 64
BB = 16


def _body(nk_ref, nv_ref, km_ref, vm_ref, gate_ref, it_ref, wb_ref,
          out_km_ref, out_vm_ref, out_read_ref):
    km = km_ref[...]
    vm = vm_ref[...]
    nv = nv_ref[...]
    it = it_ref[...]
    w = wb_ref[0, 0]
    bconf = wb_ref[0, 1]

    slot = jax.lax.broadcasted_iota(jnp.int32, (BB, N), 1)
    mask = slot <= it

    sim = jnp.sum(vm * nv[:, None, :], axis=2)
    sim = jnp.where(mask, sim, 0.0)

    m = jnp.max(sim, axis=1, keepdims=True)
    e = jnp.exp(sim - m)
    wv = e / jnp.sum(e, axis=1, keepdims=True)

    conf = jax.nn.sigmoid(sim * w + bconf)

    read_k = jnp.sum(wv[:, :, None] * km, axis=1)
    read_c = jnp.sum(wv * conf, axis=1, keepdims=True)

    scale = jax.nn.sigmoid(gate_ref[...])
    live = (it > 1).astype(jnp.float32)
    out_read_ref[:, :KD] = read_k * scale * live
    out_read_ref[:, KD:] = read_c * scale * live

    out_km_ref[:, :N, :] = km
    out_km_ref[:, N:, :] = nk_ref[...][:, None, :]
    out_vm_ref[:, :N, :] = vm
    out_vm_ref[:, N:, :] = nv[:, None, :]


def kernel(new_key, new_value, key_memory, value_memory, gate, iteration, W_conf, b_conf):
    wb = jnp.concatenate([W_conf[0], b_conf]).reshape(1, 2)
    grid = (B // BB,)
    row = lambda i: (i, 0)
    row3 = lambda i: (i, 0, 0)
    fixed = lambda i: (0, 0)
    out = pl.pallas_call(
        _body,
        grid=grid,
        in_specs=[
            pl.BlockSpec((BB, KD), row),
            pl.BlockSpec((BB, VD), row),
            pl.BlockSpec((BB, N, KD), row3),
            pl.BlockSpec((BB, N, VD), row3),
            pl.BlockSpec((BB, 1), row),
            pl.BlockSpec((BB, 1), row),
            pl.BlockSpec((1, 2), fixed),
        ],
        out_specs=[
            pl.BlockSpec((BB, N + 1, KD), row3),
            pl.BlockSpec((BB, N + 1, VD), row3),
            pl.BlockSpec((BB, KD + 1), row),
        ],
        out_shape=[
            jax.ShapeDtypeStruct((B, N + 1, KD), jnp.float32),
            jax.ShapeDtypeStruct((B, N + 1, VD), jnp.float32),
            jax.ShapeDtypeStruct((B, KD + 1), jnp.float32),
        ],
    )(new_key, new_value, key_memory, value_memory, gate, iteration, wb)
    return (out[0], out[1], out[2])


# speedup vs baseline: 1.0209x; 1.0209x over previous
import jax
import jax.numpy as jnp
from jax.experimental import pallas as pl


B, N, KD, VD = 4096, 200, 64, 64
BB = 64


def _body(nk_ref, nv_ref, km_ref, vm_ref, gate_ref, it_ref, wb_ref,
          out_km_ref, out_vm_ref, out_read_ref):
    km = km_ref[...]
    vm = vm_ref[...]
    nv = nv_ref[...]
    it = it_ref[...]
    w = wb_ref[0, 0]
    bconf = wb_ref[0, 1]

    slot = jax.lax.broadcasted_iota(jnp.int32, (BB, N), 1)
    mask = slot <= it

    sim = jax.lax.dot_general(vm, nv, (((2,), (1,)), ((0,), (0,))),
                              preferred_element_type=jnp.float32)
    sim = jnp.where(mask, sim, 0.0)

    m = jnp.max(sim, axis=1, keepdims=True)
    e = jnp.exp(sim - m)
    wv = e / jnp.sum(e, axis=1, keepdims=True)

    conf = jax.nn.sigmoid(sim * w + bconf)

    read_k = jax.lax.dot_general(wv, km, (((1,), (1,)), ((0,), (0,))),
                                 preferred_element_type=jnp.float32)
    read_c = jnp.sum(wv * conf, axis=1, keepdims=True)

    scale = jax.nn.sigmoid(gate_ref[...])
    live = (it > 1).astype(jnp.float32)
    out_read_ref[:, :KD] = read_k * scale * live
    out_read_ref[:, KD:] = read_c * scale * live

    out_km_ref[:, :N, :] = km
    out_km_ref[:, N:, :] = nk_ref[...][:, None, :]
    out_vm_ref[:, :N, :] = vm
    out_vm_ref[:, N:, :] = nv[:, None, :]


def kernel(new_key, new_value, key_memory, value_memory, gate, iteration, W_conf, b_conf):
    wb = jnp.concatenate([W_conf[0], b_conf]).reshape(1, 2)
    grid = (B // BB,)
    row = lambda i: (i, 0)
    row3 = lambda i: (i, 0, 0)
    fixed = lambda i: (0, 0)
    out = pl.pallas_call(
        _body,
        grid=grid,
        in_specs=[
            pl.BlockSpec((BB, KD), row),
            pl.BlockSpec((BB, VD), row),
            pl.BlockSpec((BB, N, KD), row3),
            pl.BlockSpec((BB, N, VD), row3),
            pl.BlockSpec((BB, 1), row),
            pl.BlockSpec((BB, 1), row),
            pl.BlockSpec((1, 2), fixed),
        ],
        out_specs=[
            pl.BlockSpec((BB, N + 1, KD), row3),
            pl.BlockSpec((BB, N + 1, VD), row3),
            pl.BlockSpec((BB, KD + 1), row),
        ],
        out_shape=[
            jax.ShapeDtypeStruct((B, N + 1, KD), jnp.float32),
            jax.ShapeDtypeStruct((B, N + 1, VD), jnp.float32),
            jax.ShapeDtypeStruct((B, KD + 1), jnp.float32),
        ],
    )(new_key, new_value, key_memory, value_memory, gate, iteration, wb)
    return (out[0], out[1], out[2])


# speedup vs baseline: 6.5972x; 6.4622x over previous
import jax
import jax.numpy as jnp
from jax.experimental import pallas as pl


B, N, KD, VD = 4096, 200, 64, 64
BL = 128


def _body(nk_ref, nv_ref, km_ref, vm_ref, gate_ref, it_ref, wb_ref,
          out_km_ref, out_vm_ref, out_read_ref):
    km = km_ref[...]
    vm = vm_ref[...]
    nv = nv_ref[...]
    it = it_ref[...]
    w = wb_ref[0, 0]
    bconf = wb_ref[0, 1]

    slot = jax.lax.broadcasted_iota(jnp.int32, (N, BL), 0)
    mask = slot <= it

    sim = jnp.sum(vm * nv[None, :, :], axis=1)
    sim = jnp.where(mask, sim, 0.0)

    m = jnp.max(sim, axis=0, keepdims=True)
    e = jnp.exp(sim - m)
    wv = e / jnp.sum(e, axis=0, keepdims=True)

    conf = jax.nn.sigmoid(sim * w + bconf)

    read_k = jnp.sum(wv[:, None, :] * km, axis=0)
    read_c = jnp.sum(wv * conf, axis=0, keepdims=True)

    scale = jax.nn.sigmoid(gate_ref[...])
    scale = scale * (it > 1).astype(jnp.float32)
    out_read_ref[:KD, :] = read_k * scale
    out_read_ref[KD:, :] = read_c * scale

    out_km_ref[:N, :, :] = km
    out_km_ref[N:, :, :] = nk_ref[...][None, :, :]
    out_vm_ref[:N, :, :] = vm
    out_vm_ref[N:, :, :] = nv[None, :, :]


def kernel(new_key, new_value, key_memory, value_memory, gate, iteration, W_conf, b_conf):
    wb = jnp.concatenate([W_conf[0], b_conf]).reshape(1, 2)
    nkT = new_key.T
    nvT = new_value.T
    kmT = jnp.transpose(key_memory, (1, 2, 0))
    vmT = jnp.transpose(value_memory, (1, 2, 0))
    gateT = gate.T
    itT = iteration.T
    grid = (B // BL,)
    col = lambda i: (0, i)
    col3 = lambda i: (0, 0, i)
    fixed = lambda i: (0, 0)
    out = pl.pallas_call(
        _body,
        grid=grid,
        in_specs=[
            pl.BlockSpec((KD, BL), col),
            pl.BlockSpec((VD, BL), col),
            pl.BlockSpec((N, KD, BL), col3),
            pl.BlockSpec((N, VD, BL), col3),
            pl.BlockSpec((1, BL), col),
            pl.BlockSpec((1, BL), col),
            pl.BlockSpec((1, 2), fixed),
        ],
        out_specs=[
            pl.BlockSpec((N + 1, KD, BL), col3),
            pl.BlockSpec((N + 1, VD, BL), col3),
            pl.BlockSpec((KD + 1, BL), col),
        ],
        out_shape=[
            jax.ShapeDtypeStruct((N + 1, KD, B), jnp.float32),
            jax.ShapeDtypeStruct((N + 1, VD, B), jnp.float32),
            jax.ShapeDtypeStruct((KD + 1, B), jnp.float32),
        ],
    )(nkT, nvT, kmT, vmT, gateT, itT, wb)
    return (jnp.transpose(out[0], (2, 0, 1)),
            jnp.transpose(out[1], (2, 0, 1)),
            out[2].T)
